# Initial kernel scaffold; baseline (speedup 1.0000x reference)
import jax
import jax.numpy as jnp
from jax.experimental import pallas as pl


def kernel(x, edge_index, set_indices, batch, ir_score, W_l, b_l, W_r, W1, b1, W2, b2, num_graphs):
    raise NotImplementedError("write your pallas kernel here")


# speedup vs baseline: 3.6507x; 3.6507x over previous
import functools

import jax
import jax.numpy as jnp
from jax import lax
from jax.experimental import pallas as pl
from jax.experimental.pallas import tpu as pltpu
from jax.experimental.pallas import tpu_sc as plsc


def _sc_gather(table, idx):
    info = plsc.get_sparse_core_info()
    nc, ns = info.num_cores, info.num_subcores
    nw = nc * ns
    B = idx.shape[0]
    D = table.shape[1]
    b_per_w = B // nw
    ch = 400
    n_it = b_per_w // ch
    mesh = plsc.VectorSubcoreMesh(core_axis_name="c", subcore_axis_name="s")

    @functools.partial(
        pl.kernel,
        mesh=mesh,
        out_type=jax.ShapeDtypeStruct((B, D), jnp.float32),
        scratch_types=[
            pltpu.VMEM((ch,), jnp.int32),
            pltpu.VMEM((ch, D), jnp.float32),
            pltpu.SemaphoreType.DMA,
        ],
    )
    def k(table_hbm, idx_hbm, out_hbm, idx_v, rows_v, sem):
        wid = lax.axis_index("s") * nc + lax.axis_index("c")

        def it(t, carry):
            base = wid * b_per_w + t * ch
            pltpu.sync_copy(idx_hbm.at[pl.ds(base, ch)], idx_v)
            pltpu.async_copy(table_hbm.at[idx_v], rows_v, sem).wait()
            pltpu.sync_copy(rows_v, out_hbm.at[pl.ds(base, ch)])
            return carry

        lax.fori_loop(0, n_it, it, 0)

    return k(table, idx)


def kernel(x, edge_index, set_indices, batch, ir_score, W_l, b_l, W_r,
           W1, b1, W2, b2, num_graphs):
    N, D = x.shape
    E = edge_index.shape[1]
    G = set_indices.shape[0]
    GS = W_l.shape[0]
    H1 = W1.shape[0]
    NB = W2.shape[0]
    P = 2 * G

    src = edge_index[0]
    dst = edge_index[1]

    xsrc = _sc_gather(x, src)

    EC = 512
    nblk = E // EC
    dst3 = dst.reshape(nblk, 1, EC)

    setf = jnp.broadcast_to(
        set_indices.T.reshape(P, 1).astype(jnp.int32), (P, 128))

    NPAD = 10240
    batch2 = jnp.concatenate(
        [batch.astype(jnp.int32),
         jnp.full((NPAD - N,), 2 ** 20, jnp.int32)]).reshape(NPAD // 128, 128)
    n_brow = NPAD // 128

    w1a = jnp.zeros((256, GS), jnp.float32).at[:H1].set(W1[:, :GS])
    w1b = jnp.zeros((256, GS), jnp.float32).at[:H1].set(W1[:, GS:])
    b1p = jnp.broadcast_to(
        jnp.zeros((256,), jnp.float32).at[:H1].set(b1)[None, :], (8, 256))
    w2p = jnp.zeros((128, 256), jnp.float32).at[:NB, :H1].set(W2)
    b2p = jnp.broadcast_to(
        jnp.zeros((128,), jnp.float32).at[:NB].set(b2)[None, :], (8, 128))
    blp = jnp.broadcast_to(b_l[None, :], (8, GS))

    xc = 2000
    n_xc = N // xc

    def body(dst_ref, xsrc_ref, batch_ref, setf_ref, x_ref, wl_ref, bl_ref,
             wr_ref, w1a_ref, w1b_ref, b1_ref, w2_ref, b2_ref, out_ref,
             ids_s, agg_s, deg_s):
        i = pl.program_id(0)

        @pl.when(i == 0)
        def _init():
            p = lax.broadcasted_iota(jnp.int32, (P, 1), 0)
            gid = p % G

            def bf(j, acc):
                row = batch_ref[pl.ds(j, 1), :]
                return acc + jnp.sum((row < gid).astype(jnp.int32),
                                     axis=1, keepdims=True)

            bases = lax.fori_loop(0, n_brow, bf,
                                  jnp.zeros((P, 1), jnp.int32))
            ids = jnp.clip(bases + setf_ref[:, 0:1], 0, N - 1)
            ids_s[...] = jnp.broadcast_to(ids, (P, 128))
            agg_s[...] = jnp.zeros((P, 128), jnp.float32)
            deg_s[...] = jnp.zeros((P, 128), jnp.float32)

        ids = ids_s[:, 0:1]
        m = (ids == dst_ref[0]).astype(jnp.float32)
        agg_s[...] += jnp.dot(m, xsrc_ref[...],
                              preferred_element_type=jnp.float32)
        deg_s[...] += jnp.broadcast_to(
            jnp.sum(m, axis=1, keepdims=True), (P, 128))

        @pl.when(i == nblk - 1)
        def _decode():
            dg = lambda a, b: lax.dot_general(
                a, b, (((1,), (1,)), ((), ())),
                preferred_element_type=jnp.float32)
            ids2 = ids_s[:, 0:1]
            deg = jnp.maximum(deg_s[:, 0:1], 1.0)
            mean = agg_s[...] / deg

            def xf(j, acc):
                base = j * xc
                rows = x_ref[pl.ds(base, xc), :]
                ridx = lax.broadcasted_iota(jnp.int32, (1, xc), 1) + base
                mm = (ids2 == ridx).astype(jnp.float32)
                return acc + jnp.dot(mm, rows,
                                     preferred_element_type=jnp.float32)

            x512 = lax.fori_loop(0, n_xc, xf,
                                 jnp.zeros((P, 128), jnp.float32))
            h = dg(mean, wl_ref[...]) + bl_ref[0:1, :] + dg(x512, wr_ref[...])
            h = jnp.maximum(h, 0.0)
            e0 = h[0:G, :]
            e1 = h[G:P, :]
            l1 = dg(e0, w1a_ref[...]) + dg(e1, w1b_ref[...]) + b1_ref[0:1, :]
            l1 = jnp.maximum(l1, 0.0)
            out_ref[...] = dg(l1, w2_ref[...]) + b2_ref[0:1, :]

    out = pl.pallas_call(
        body,
        grid=(nblk,),
        in_specs=[
            pl.BlockSpec((1, 1, EC), lambda i: (i, 0, 0)),
            pl.BlockSpec((EC, D), lambda i: (i, 0)),
            pl.BlockSpec((n_brow, 128), lambda i: (0, 0)),
            pl.BlockSpec((P, 128), lambda i: (0, 0)),
            pl.BlockSpec((N, D), lambda i: (0, 0)),
            pl.BlockSpec((GS, D), lambda i: (0, 0)),
            pl.BlockSpec((8, GS), lambda i: (0, 0)),
            pl.BlockSpec((GS, D), lambda i: (0, 0)),
            pl.BlockSpec((256, GS), lambda i: (0, 0)),
            pl.BlockSpec((256, GS), lambda i: (0, 0)),
            pl.BlockSpec((8, 256), lambda i: (0, 0)),
            pl.BlockSpec((128, 256), lambda i: (0, 0)),
            pl.BlockSpec((8, 128), lambda i: (0, 0)),
        ],
        out_specs=pl.BlockSpec((G, 128), lambda i: (0, 0)),
        out_shape=jax.ShapeDtypeStruct((G, 128), jnp.float32),
        scratch_shapes=[
            pltpu.VMEM((P, 128), jnp.int32),
            pltpu.VMEM((P, 128), jnp.float32),
            pltpu.VMEM((P, 128), jnp.float32),
        ],
    )(dst3, xsrc, batch2, setf, x, W_l, blp, W_r, w1a, w1b, b1p, w2p, b2p)

    return out[:, :NB]


# speedup vs baseline: 3.7369x; 1.0236x over previous
import functools

import jax
import jax.numpy as jnp
from jax import lax
from jax.experimental import pallas as pl
from jax.experimental.pallas import tpu as pltpu
from jax.experimental.pallas import tpu_sc as plsc


def _sc_gather(table, idx):
    info = plsc.get_sparse_core_info()
    nc, ns = info.num_cores, info.num_subcores
    nw = nc * ns
    B = idx.shape[0]
    D = table.shape[1]
    b_per_w = B // nw
    ch = 200
    n_it = b_per_w // (2 * ch)
    mesh = plsc.VectorSubcoreMesh(core_axis_name="c", subcore_axis_name="s")

    @functools.partial(
        pl.kernel,
        mesh=mesh,
        out_type=jax.ShapeDtypeStruct((B, D), jnp.float32),
        scratch_types=[
            pltpu.VMEM((b_per_w,), jnp.int32),
            pltpu.VMEM((ch, D), jnp.float32),
            pltpu.VMEM((ch, D), jnp.float32),
            pltpu.SemaphoreType.DMA,
            pltpu.SemaphoreType.DMA,
        ],
    )
    def k(table_hbm, idx_hbm, out_hbm, idx_v, r0, r1, sem0, sem1):
        wid = lax.axis_index("s") * nc + lax.axis_index("c")
        wbase = wid * b_per_w
        pltpu.sync_copy(idx_hbm.at[pl.ds(wbase, b_per_w)], idx_v)

        def it(t, carry):
            o0 = 2 * t * ch
            o1 = o0 + ch
            c0 = pltpu.async_copy(
                table_hbm.at[idx_v.at[pl.ds(o0, ch)]], r0, sem0)
            c1 = pltpu.async_copy(
                table_hbm.at[idx_v.at[pl.ds(o1, ch)]], r1, sem1)
            c0.wait()
            pltpu.sync_copy(r0, out_hbm.at[pl.ds(wbase + o0, ch)])
            c1.wait()
            pltpu.sync_copy(r1, out_hbm.at[pl.ds(wbase + o1, ch)])
            return carry

        lax.fori_loop(0, n_it, it, 0)

    return k(table, idx)


def kernel(x, edge_index, set_indices, batch, ir_score, W_l, b_l, W_r,
           W1, b1, W2, b2, num_graphs):
    N, D = x.shape
    E = edge_index.shape[1]
    G = set_indices.shape[0]
    GS = W_l.shape[0]
    H1 = W1.shape[0]
    NB = W2.shape[0]
    P = 2 * G

    src = edge_index[0]
    dst = edge_index[1]

    xsrc = _sc_gather(x, src)

    EC = 512
    nblk = E // EC
    dst3 = dst.reshape(nblk, 1, EC)

    setf = jnp.broadcast_to(
        set_indices.T.reshape(P, 1).astype(jnp.int32), (P, 128))

    NPAD = 10240
    batch2 = jnp.concatenate(
        [batch.astype(jnp.int32),
         jnp.full((NPAD - N,), 2 ** 20, jnp.int32)]).reshape(NPAD // 128, 128)
    n_brow = NPAD // 128

    w1a = jnp.zeros((256, GS), jnp.float32).at[:H1].set(W1[:, :GS])
    w1b = jnp.zeros((256, GS), jnp.float32).at[:H1].set(W1[:, GS:])
    b1p = jnp.broadcast_to(
        jnp.zeros((256,), jnp.float32).at[:H1].set(b1)[None, :], (8, 256))
    w2p = jnp.zeros((128, 256), jnp.float32).at[:NB, :H1].set(W2)
    b2p = jnp.broadcast_to(
        jnp.zeros((128,), jnp.float32).at[:NB].set(b2)[None, :], (8, 128))
    blp = jnp.broadcast_to(b_l[None, :], (8, GS))

    xc = 2000
    n_xc = N // xc

    def body(dst_ref, xsrc_ref, batch_ref, setf_ref, x_ref, wl_ref, bl_ref,
             wr_ref, w1a_ref, w1b_ref, b1_ref, w2_ref, b2_ref, out_ref,
             ids_s, agg_s, deg_s):
        i = pl.program_id(0)

        @pl.when(i == 0)
        def _init():
            p = lax.broadcasted_iota(jnp.int32, (P, 1), 0)
            gid = p % G

            def bf(j, acc):
                row = batch_ref[pl.ds(j, 1), :]
                return acc + jnp.sum((row < gid).astype(jnp.int32),
                                     axis=1, keepdims=True)

            bases = lax.fori_loop(0, n_brow, bf,
                                  jnp.zeros((P, 1), jnp.int32))
            ids = jnp.clip(bases + setf_ref[:, 0:1], 0, N - 1)
            ids_s[...] = jnp.broadcast_to(ids, (P, 128))
            agg_s[...] = jnp.zeros((P, 128), jnp.float32)
            deg_s[...] = jnp.zeros((P, 128), jnp.float32)

        ids = ids_s[:, 0:1]
        m = (ids == dst_ref[0]).astype(jnp.float32)
        agg_s[...] += jnp.dot(m, xsrc_ref[...],
                              preferred_element_type=jnp.float32)
        deg_s[...] += jnp.broadcast_to(
            jnp.sum(m, axis=1, keepdims=True), (P, 128))

        @pl.when(i == nblk - 1)
        def _decode():
            dg = lambda a, b: lax.dot_general(
                a, b, (((1,), (1,)), ((), ())),
                preferred_element_type=jnp.float32)
            ids2 = ids_s[:, 0:1]
            deg = jnp.maximum(deg_s[:, 0:1], 1.0)
            mean = agg_s[...] / deg

            def xf(j, acc):
                base = j * xc
                rows = x_ref[pl.ds(base, xc), :]
                ridx = lax.broadcasted_iota(jnp.int32, (1, xc), 1) + base
                mm = (ids2 == ridx).astype(jnp.float32)
                return acc + jnp.dot(mm, rows,
                                     preferred_element_type=jnp.float32)

            x512 = lax.fori_loop(0, n_xc, xf,
                                 jnp.zeros((P, 128), jnp.float32))
            h = dg(mean, wl_ref[...]) + bl_ref[0:1, :] + dg(x512, wr_ref[...])
            h = jnp.maximum(h, 0.0)
            e0 = h[0:G, :]
            e1 = h[G:P, :]
            l1 = dg(e0, w1a_ref[...]) + dg(e1, w1b_ref[...]) + b1_ref[0:1, :]
            l1 = jnp.maximum(l1, 0.0)
            out_ref[...] = dg(l1, w2_ref[...]) + b2_ref[0:1, :]

    out = pl.pallas_call(
        body,
        grid=(nblk,),
        in_specs=[
            pl.BlockSpec((1, 1, EC), lambda i: (i, 0, 0)),
            pl.BlockSpec((EC, D), lambda i: (i, 0)),
            pl.BlockSpec((n_brow, 128), lambda i: (0, 0)),
            pl.BlockSpec((P, 128), lambda i: (0, 0)),
            pl.BlockSpec((N, D), lambda i: (0, 0)),
            pl.BlockSpec((GS, D), lambda i: (0, 0)),
            pl.BlockSpec((8, GS), lambda i: (0, 0)),
            pl.BlockSpec((GS, D), lambda i: (0, 0)),
            pl.BlockSpec((256, GS), lambda i: (0, 0)),
            pl.BlockSpec((256, GS), lambda i: (0, 0)),
            pl.BlockSpec((8, 256), lambda i: (0, 0)),
            pl.BlockSpec((128, 256), lambda i: (0, 0)),
            pl.BlockSpec((8, 128), lambda i: (0, 0)),
        ],
        out_specs=pl.BlockSpec((G, 128), lambda i: (0, 0)),
        out_shape=jax.ShapeDtypeStruct((G, 128), jnp.float32),
        scratch_shapes=[
            pltpu.VMEM((P, 128), jnp.int32),
            pltpu.VMEM((P, 128), jnp.float32),
            pltpu.VMEM((P, 128), jnp.float32),
        ],
    )(dst3, xsrc, batch2, setf, x, W_l, blp, W_r, w1a, w1b, b1p, w2p, b2p)

    return out[:, :NB]


# speedup vs baseline: 6.4512x; 1.7264x over previous
import functools

import jax
import jax.numpy as jnp
from jax import lax
from jax.experimental import pallas as pl
from jax.experimental.pallas import tpu as pltpu
from jax.experimental.pallas import tpu_sc as plsc


def _sc_gather(table, idx):
    info = plsc.get_sparse_core_info()
    nc, ns = info.num_cores, info.num_subcores
    nw = nc * ns
    B = idx.shape[0]
    D = table.shape[1]
    b_per_w = B // nw
    ch = 200
    n_it = b_per_w // (2 * ch)
    mesh = plsc.VectorSubcoreMesh(core_axis_name="c", subcore_axis_name="s")

    @functools.partial(
        pl.kernel,
        mesh=mesh,
        out_type=jax.ShapeDtypeStruct((B, D), jnp.float32),
        scratch_types=[
            pltpu.VMEM((b_per_w,), jnp.int32),
            pltpu.VMEM((ch, D), jnp.float32),
            pltpu.VMEM((ch, D), jnp.float32),
            pltpu.SemaphoreType.DMA,
            pltpu.SemaphoreType.DMA,
        ],
    )
    def k(table_hbm, idx_hbm, out_hbm, idx_v, r0, r1, sem0, sem1):
        wid = lax.axis_index("s") * nc + lax.axis_index("c")
        wbase = wid * b_per_w
        pltpu.sync_copy(idx_hbm.at[pl.ds(wbase, b_per_w)], idx_v)

        def it(t, carry):
            o0 = 2 * t * ch
            o1 = o0 + ch
            c0 = pltpu.async_copy(
                table_hbm.at[idx_v.at[pl.ds(o0, ch)]], r0, sem0)
            c1 = pltpu.async_copy(
                table_hbm.at[idx_v.at[pl.ds(o1, ch)]], r1, sem1)
            c0.wait()
            pltpu.sync_copy(r0, out_hbm.at[pl.ds(wbase + o0, ch)])
            c1.wait()
            pltpu.sync_copy(r1, out_hbm.at[pl.ds(wbase + o1, ch)])
            return carry

        lax.fori_loop(0, n_it, it, 0)

    return k(table, idx)


def kernel(x, edge_index, set_indices, batch, ir_score, W_l, b_l, W_r,
           W1, b1, W2, b2, num_graphs):
    N, D = x.shape
    E = edge_index.shape[1]
    G = set_indices.shape[0]
    GS = W_l.shape[0]
    H1 = W1.shape[0]
    NB = W2.shape[0]
    P = 2 * G

    src = edge_index[0]
    dst = edge_index[1]

    xsrc = _sc_gather(x, src)

    EC = 2560
    nblk = E // EC
    dst3 = dst.reshape(nblk, 1, EC)

    setf = jnp.broadcast_to(
        set_indices.T.reshape(P, 1).astype(jnp.int32), (P, 128))

    NPAD = 10240
    batch2 = jnp.concatenate(
        [batch.astype(jnp.int32),
         jnp.full((NPAD - N,), 2 ** 20, jnp.int32)]).reshape(NPAD // 128, 128)
    n_brow = NPAD // 128

    w1a = jnp.zeros((256, GS), jnp.float32).at[:H1].set(W1[:, :GS])
    w1b = jnp.zeros((256, GS), jnp.float32).at[:H1].set(W1[:, GS:])
    b1p = jnp.broadcast_to(
        jnp.zeros((256,), jnp.float32).at[:H1].set(b1)[None, :], (8, 256))
    w2p = jnp.zeros((128, 256), jnp.float32).at[:NB, :H1].set(W2)
    b2p = jnp.broadcast_to(
        jnp.zeros((128,), jnp.float32).at[:NB].set(b2)[None, :], (8, 128))
    blp = jnp.broadcast_to(b_l[None, :], (8, GS))

    xc = 2000
    n_xc = N // xc

    def body(dst_ref, xsrc_ref, batch_ref, setf_ref, x_ref, wl_ref, bl_ref,
             wr_ref, w1a_ref, w1b_ref, b1_ref, w2_ref, b2_ref, out_ref,
             ids_s, agg_s, deg_s):
        i = pl.program_id(0)

        @pl.when(i == 0)
        def _init():
            p = lax.broadcasted_iota(jnp.int32, (P, 1), 0)
            gid = p % G

            def bf(j, acc):
                row = batch_ref[pl.ds(j, 1), :]
                return acc + jnp.sum((row < gid).astype(jnp.int32),
                                     axis=1, keepdims=True)

            bases = lax.fori_loop(0, n_brow, bf,
                                  jnp.zeros((P, 1), jnp.int32))
            ids = jnp.clip(bases + setf_ref[:, 0:1], 0, N - 1)
            ids_s[...] = jnp.broadcast_to(ids, (P, 128))
            agg_s[...] = jnp.zeros((P, 128), jnp.float32)
            deg_s[...] = jnp.zeros((P, 128), jnp.float32)

        ids = ids_s[:, 0:1]
        m = (ids == dst_ref[0]).astype(jnp.float32)
        agg_s[...] += jnp.dot(m, xsrc_ref[...],
                              preferred_element_type=jnp.float32)
        deg_s[...] += jnp.broadcast_to(
            jnp.sum(m, axis=1, keepdims=True), (P, 128))

        @pl.when(i == nblk - 1)
        def _decode():
            dg = lambda a, b: lax.dot_general(
                a, b, (((1,), (1,)), ((), ())),
                preferred_element_type=jnp.float32)
            ids2 = ids_s[:, 0:1]
            deg = jnp.maximum(deg_s[:, 0:1], 1.0)
            mean = agg_s[...] / deg

            def xf(j, acc):
                base = j * xc
                rows = x_ref[pl.ds(base, xc), :]
                ridx = lax.broadcasted_iota(jnp.int32, (1, xc), 1) + base
                mm = (ids2 == ridx).astype(jnp.float32)
                return acc + jnp.dot(mm, rows,
                                     preferred_element_type=jnp.float32)

            x512 = lax.fori_loop(0, n_xc, xf,
                                 jnp.zeros((P, 128), jnp.float32))
            h = dg(mean, wl_ref[...]) + bl_ref[0:1, :] + dg(x512, wr_ref[...])
            h = jnp.maximum(h, 0.0)
            e0 = h[0:G, :]
            e1 = h[G:P, :]
            l1 = dg(e0, w1a_ref[...]) + dg(e1, w1b_ref[...]) + b1_ref[0:1, :]
            l1 = jnp.maximum(l1, 0.0)
            out_ref[...] = dg(l1, w2_ref[...]) + b2_ref[0:1, :]

    out = pl.pallas_call(
        body,
        grid=(nblk,),
        in_specs=[
            pl.BlockSpec((1, 1, EC), lambda i: (i, 0, 0)),
            pl.BlockSpec((EC, D), lambda i: (i, 0)),
            pl.BlockSpec((n_brow, 128), lambda i: (0, 0)),
            pl.BlockSpec((P, 128), lambda i: (0, 0)),
            pl.BlockSpec((N, D), lambda i: (0, 0)),
            pl.BlockSpec((GS, D), lambda i: (0, 0)),
            pl.BlockSpec((8, GS), lambda i: (0, 0)),
            pl.BlockSpec((GS, D), lambda i: (0, 0)),
            pl.BlockSpec((256, GS), lambda i: (0, 0)),
            pl.BlockSpec((256, GS), lambda i: (0, 0)),
            pl.BlockSpec((8, 256), lambda i: (0, 0)),
            pl.BlockSpec((128, 256), lambda i: (0, 0)),
            pl.BlockSpec((8, 128), lambda i: (0, 0)),
        ],
        out_specs=pl.BlockSpec((G, 128), lambda i: (0, 0)),
        out_shape=jax.ShapeDtypeStruct((G, 128), jnp.float32),
        scratch_shapes=[
            pltpu.VMEM((P, 128), jnp.int32),
            pltpu.VMEM((P, 128), jnp.float32),
            pltpu.VMEM((P, 128), jnp.float32),
        ],
    )(dst3, xsrc, batch2, setf, x, W_l, blp, W_r, w1a, w1b, b1p, w2p, b2p)

    return out[:, :NB]


# speedup vs baseline: 6.6792x; 1.0353x over previous
import functools

import jax
import jax.numpy as jnp
from jax import lax
from jax.experimental import pallas as pl
from jax.experimental.pallas import tpu as pltpu
from jax.experimental.pallas import tpu_sc as plsc


def _sc_gather(table, idx):
    info = plsc.get_sparse_core_info()
    nc, ns = info.num_cores, info.num_subcores
    nw = nc * ns
    B = idx.shape[0]
    D = table.shape[1]
    b_per_w = B // nw
    ch = 200
    n_it = b_per_w // (2 * ch)
    mesh = plsc.VectorSubcoreMesh(core_axis_name="c", subcore_axis_name="s")

    @functools.partial(
        pl.kernel,
        mesh=mesh,
        out_type=jax.ShapeDtypeStruct((B, D), jnp.float32),
        scratch_types=[
            pltpu.VMEM((b_per_w,), jnp.int32),
            pltpu.VMEM((ch, D), jnp.float32),
            pltpu.VMEM((ch, D), jnp.float32),
            pltpu.SemaphoreType.DMA,
            pltpu.SemaphoreType.DMA,
        ],
    )
    def k(table_hbm, idx_hbm, out_hbm, idx_v, r0, r1, sem0, sem1):
        wid = lax.axis_index("s") * nc + lax.axis_index("c")
        wbase = wid * b_per_w
        pltpu.sync_copy(idx_hbm.at[pl.ds(wbase, b_per_w)], idx_v)

        def it(t, carry):
            o0 = 2 * t * ch
            o1 = o0 + ch
            c0 = pltpu.async_copy(
                table_hbm.at[idx_v.at[pl.ds(o0, ch)]], r0, sem0)
            c1 = pltpu.async_copy(
                table_hbm.at[idx_v.at[pl.ds(o1, ch)]], r1, sem1)
            c0.wait()
            pltpu.sync_copy(r0, out_hbm.at[pl.ds(wbase + o0, ch)])
            c1.wait()
            pltpu.sync_copy(r1, out_hbm.at[pl.ds(wbase + o1, ch)])
            return carry

        lax.fori_loop(0, n_it, it, 0)

    return k(table, idx)


def kernel(x, edge_index, set_indices, batch, ir_score, W_l, b_l, W_r,
           W1, b1, W2, b2, num_graphs):
    N, D = x.shape
    E = edge_index.shape[1]
    G = set_indices.shape[0]
    GS = W_l.shape[0]
    H1 = W1.shape[0]
    NB = W2.shape[0]
    P = 2 * G

    src = edge_index[0]
    dst = edge_index[1]

    xsrc = _sc_gather(x, src)

    EC = 6400
    nblk = E // EC
    dst3 = dst.reshape(nblk, 1, EC)

    setf = jnp.broadcast_to(
        set_indices.T.reshape(P, 1).astype(jnp.int32), (P, 128))

    NPAD = 10240
    batch2 = jnp.concatenate(
        [batch.astype(jnp.int32),
         jnp.full((NPAD - N,), 2 ** 20, jnp.int32)]).reshape(NPAD // 128, 128)
    n_brow = NPAD // 128

    w1a = jnp.zeros((256, GS), jnp.float32).at[:H1].set(W1[:, :GS])
    w1b = jnp.zeros((256, GS), jnp.float32).at[:H1].set(W1[:, GS:])
    b1p = jnp.broadcast_to(
        jnp.zeros((256,), jnp.float32).at[:H1].set(b1)[None, :], (8, 256))
    w2p = jnp.zeros((128, 256), jnp.float32).at[:NB, :H1].set(W2)
    b2p = jnp.broadcast_to(
        jnp.zeros((128,), jnp.float32).at[:NB].set(b2)[None, :], (8, 128))
    blp = jnp.broadcast_to(b_l[None, :], (8, GS))

    xc = 2000
    n_xc = N // xc

    def body(dst_ref, xsrc_ref, batch_ref, setf_ref, x_ref, wl_ref, bl_ref,
             wr_ref, w1a_ref, w1b_ref, b1_ref, w2_ref, b2_ref, out_ref,
             ids_s, agg_s, deg_s):
        i = pl.program_id(0)

        @pl.when(i == 0)
        def _init():
            p = lax.broadcasted_iota(jnp.int32, (P, 1), 0)
            gid = p % G

            def bf(j, acc):
                row = batch_ref[pl.ds(j, 1), :]
                return acc + jnp.sum((row < gid).astype(jnp.int32),
                                     axis=1, keepdims=True)

            bases = lax.fori_loop(0, n_brow, bf,
                                  jnp.zeros((P, 1), jnp.int32))
            ids = jnp.clip(bases + setf_ref[:, 0:1], 0, N - 1)
            ids_s[...] = jnp.broadcast_to(ids, (P, 128))
            agg_s[...] = jnp.zeros((P, 128), jnp.float32)
            deg_s[...] = jnp.zeros((P, 128), jnp.float32)

        ids = ids_s[:, 0:1]
        m = (ids == dst_ref[0]).astype(jnp.float32)
        agg_s[...] += jnp.dot(m, xsrc_ref[...],
                              preferred_element_type=jnp.float32)
        deg_s[...] += jnp.broadcast_to(
            jnp.sum(m, axis=1, keepdims=True), (P, 128))

        @pl.when(i == nblk - 1)
        def _decode():
            dg = lambda a, b: lax.dot_general(
                a, b, (((1,), (1,)), ((), ())),
                preferred_element_type=jnp.float32)
            ids2 = ids_s[:, 0:1]
            deg = jnp.maximum(deg_s[:, 0:1], 1.0)
            mean = agg_s[...] / deg

            def xf(j, acc):
                base = j * xc
                rows = x_ref[pl.ds(base, xc), :]
                ridx = lax.broadcasted_iota(jnp.int32, (1, xc), 1) + base
                mm = (ids2 == ridx).astype(jnp.float32)
                return acc + jnp.dot(mm, rows,
                                     preferred_element_type=jnp.float32)

            x512 = lax.fori_loop(0, n_xc, xf,
                                 jnp.zeros((P, 128), jnp.float32))
            h = dg(mean, wl_ref[...]) + bl_ref[0:1, :] + dg(x512, wr_ref[...])
            h = jnp.maximum(h, 0.0)
            e0 = h[0:G, :]
            e1 = h[G:P, :]
            l1 = dg(e0, w1a_ref[...]) + dg(e1, w1b_ref[...]) + b1_ref[0:1, :]
            l1 = jnp.maximum(l1, 0.0)
            out_ref[...] = dg(l1, w2_ref[...]) + b2_ref[0:1, :]

    out = pl.pallas_call(
        body,
        grid=(nblk,),
        in_specs=[
            pl.BlockSpec((1, 1, EC), lambda i: (i, 0, 0)),
            pl.BlockSpec((EC, D), lambda i: (i, 0)),
            pl.BlockSpec((n_brow, 128), lambda i: (0, 0)),
            pl.BlockSpec((P, 128), lambda i: (0, 0)),
            pl.BlockSpec((N, D), lambda i: (0, 0)),
            pl.BlockSpec((GS, D), lambda i: (0, 0)),
            pl.BlockSpec((8, GS), lambda i: (0, 0)),
            pl.BlockSpec((GS, D), lambda i: (0, 0)),
            pl.BlockSpec((256, GS), lambda i: (0, 0)),
            pl.BlockSpec((256, GS), lambda i: (0, 0)),
            pl.BlockSpec((8, 256), lambda i: (0, 0)),
            pl.BlockSpec((128, 256), lambda i: (0, 0)),
            pl.BlockSpec((8, 128), lambda i: (0, 0)),
        ],
        out_specs=pl.BlockSpec((G, 128), lambda i: (0, 0)),
        out_shape=jax.ShapeDtypeStruct((G, 128), jnp.float32),
        scratch_shapes=[
            pltpu.VMEM((P, 128), jnp.int32),
            pltpu.VMEM((P, 128), jnp.float32),
            pltpu.VMEM((P, 128), jnp.float32),
        ],
    )(dst3, xsrc, batch2, setf, x, W_l, blp, W_r, w1a, w1b, b1p, w2p, b2p)

    return out[:, :NB]
